# baseline (speedup 1.0000x reference)
import dataclasses
import functools

import jax
import jax.numpy as jnp
from jax import lax
from jax.experimental import pallas as pl
from jax.experimental.pallas import tpu as pltpu
from jax.experimental.pallas import tpu_sc as plsc

N = 100000
NP = 102400
P = 500
NPP = 200
E = 1600000
MN = 500

NC = 2
NS = 16
NW = NC * NS
EW_PER = E // NW
CHUNK = 2000
SLN = NP // NS
NA = 100096
SLA = NA // NS
CH1 = 400
CH2 = 400
EW_MAIN = (EW_PER // (2 * CH2)) * (2 * CH2)
assert EW_PER - EW_MAIN == CH2

_mesh = plsc.VectorSubcoreMesh(core_axis_name="c", subcore_axis_name="s")

_sc_params = pltpu.CompilerParams(needs_layout_passes=False,
                                  use_tc_tiling_on_sc=False)


@functools.partial(
    pl.kernel,
    out_type=(jax.ShapeDtypeStruct((NC, NP), jnp.float32),
              jax.ShapeDtypeStruct((NC, NP), jnp.float32)),
    mesh=_mesh,
    scratch_types=[
        pltpu.VMEM((CHUNK,), jnp.int32),
        pltpu.VMEM((CHUNK,), jnp.int32),
        pltpu.VMEM((CHUNK,), jnp.float32),
        pltpu.VMEM_SHARED((NP,), jnp.float32),
        pltpu.VMEM_SHARED((NP,), jnp.float32),
    ],
)
def _deg_kernel(src_hbm, dst_hbm, zeros_hbm, dego_hbm, degi_hbm,
                src_v, dst_v, ones_v, acco_s, acci_s):
    cid = lax.axis_index("c")
    sid = lax.axis_index("s")
    wid = cid * NS + sid

    @pl.loop(0, CHUNK, step=16)
    def _(i):
        ones_v[pl.ds(pl.multiple_of(i, 16), 16)] = jnp.ones((16,), jnp.float32)

    pltpu.sync_copy(zeros_hbm.at[pl.ds(sid * SLN, SLN)], acco_s.at[pl.ds(sid * SLN, SLN)])
    pltpu.sync_copy(zeros_hbm.at[pl.ds(sid * SLN, SLN)], acci_s.at[pl.ds(sid * SLN, SLN)])
    plsc.subcore_barrier()

    @pl.loop(0, EW_PER, step=CHUNK)
    def _(k):
        base = wid * EW_PER + k
        pltpu.sync_copy(src_hbm.at[pl.ds(base, CHUNK)], src_v)
        pltpu.sync_copy(dst_hbm.at[pl.ds(base, CHUNK)], dst_v)
        pltpu.sync_copy(ones_v, acco_s.at[src_v], add=True)
        pltpu.sync_copy(ones_v, acci_s.at[dst_v], add=True)

    plsc.subcore_barrier()
    pltpu.sync_copy(acco_s.at[pl.ds(sid * SLN, SLN)], dego_hbm.at[cid, pl.ds(sid * SLN, SLN)])
    pltpu.sync_copy(acci_s.at[pl.ds(sid * SLN, SLN)], degi_hbm.at[cid, pl.ds(sid * SLN, SLN)])


def _scale_rows(rows_v, e3_v, iota16, F, chunk):
    @pl.loop(0, chunk, step=16)
    def _(j):
        e16 = jnp.full((16,), j, jnp.int32) + iota16
        w16 = plsc.bitcast(e3_v[2, pl.ds(pl.multiple_of(j, 16), 16)],
                           jnp.float32)
        for f in range(F):
            fv = jnp.full((16,), f, jnp.int32)
            v = plsc.load_gather(rows_v, [e16, fv])
            plsc.store_scatter(rows_v, [e16, fv], v * w16)


def _make_agg_kernel(F, n_chunks):
    @functools.partial(
        pl.kernel,
        out_type=tuple(jax.ShapeDtypeStruct((NC, NA, F), jnp.float32)
                       for _ in range(n_chunks)),
        mesh=_mesh,
        scratch_types=[
            pltpu.VMEM((3, CH2), jnp.int32), pltpu.VMEM((3, CH2), jnp.int32),
            pltpu.VMEM((CH2, F), jnp.float32), pltpu.VMEM((CH2, F), jnp.float32),
            pltpu.VMEM_SHARED((NA, F), jnp.float32),
            pltpu.SemaphoreType.DMA, pltpu.SemaphoreType.DMA,
            pltpu.SemaphoreType.DMA, pltpu.SemaphoreType.DMA,
        ],
        compiler_params=_sc_params,
    )
    def _agg(*refs):
        xs_list = refs[:n_chunks]
        e3_hbm, zeros_hbm = refs[n_chunks:n_chunks + 2]
        out_list = refs[n_chunks + 2:2 * n_chunks + 2]
        (e30, e31, rows0, rows1, acc_s, g0, g1, sc0, sc1) = refs[2 * n_chunks + 2:]
        sets = ((e30, rows0, g0, sc0), (e31, rows1, g1, sc1))

        cid = lax.axis_index("c")
        sid = lax.axis_index("s")
        wid = cid * NS + sid
        iota16 = lax.iota(jnp.int32, 16)

        for c in range(n_chunks):
            pltpu.sync_copy(zeros_hbm, acc_s.at[pl.ds(sid * SLA, SLA)])
            plsc.subcore_barrier()

            @pl.loop(0, EW_MAIN, step=2 * CH2)
            def _(k, c=c):
                idx_d = []
                for si, (e3_v, rows_v, g_sem, sc_sem) in enumerate(sets):
                    @pl.when(k > 0)
                    def _():
                        pltpu.make_async_copy(rows_v, acc_s.at[e3_v.at[1]],
                                              sc_sem).wait()
                    base = wid * EW_PER + k + si * CH2
                    idx_d.append(pltpu.async_copy(
                        e3_hbm.at[:, pl.ds(base, CH2)], e3_v, g_sem))
                gat_d = []
                for si, (e3_v, rows_v, g_sem, sc_sem) in enumerate(sets):
                    idx_d[si].wait()
                    gat_d.append(pltpu.async_copy(
                        xs_list[c].at[e3_v.at[0]], rows_v, g_sem))
                for si, (e3_v, rows_v, g_sem, sc_sem) in enumerate(sets):
                    gat_d[si].wait()
                    _scale_rows(rows_v, e3_v, iota16, F, CH2)
                    pltpu.async_copy(rows_v, acc_s.at[e3_v.at[1]], sc_sem,
                                     add=True)

            e3_v, rows_v, g_sem, sc_sem = sets[0]
            pltpu.make_async_copy(rows_v, acc_s.at[e3_v.at[1]], sc_sem).wait()
            base = wid * EW_PER + EW_MAIN
            pltpu.async_copy(e3_hbm.at[:, pl.ds(base, CH2)], e3_v, g_sem).wait()
            pltpu.async_copy(xs_list[c].at[e3_v.at[0]], rows_v, g_sem).wait()
            _scale_rows(rows_v, e3_v, iota16, F, CH2)
            pltpu.async_copy(rows_v, acc_s.at[e3_v.at[1]], sc_sem, add=True)

            for (e3_v, rows_v, g_sem, sc_sem) in sets:
                pltpu.make_async_copy(rows_v, acc_s.at[e3_v.at[1]], sc_sem).wait()

            plsc.subcore_barrier()
            pltpu.sync_copy(acc_s.at[pl.ds(sid * SLA, SLA)],
                            out_list[c].at[cid, pl.ds(sid * SLA, SLA)])

    return _agg


_agg16x4_kernel = _make_agg_kernel(16, 4)
_agg16x2_kernel = _make_agg_kernel(16, 2)


def _lr(t):
    return jnp.where(t >= 0.0, t, 0.01 * t)


def _rs(d0, d1):
    return lax.rsqrt(jnp.maximum(d0 + d1, 1.0))


def _tc_xs_body(pf_ref, do0_ref, do1_ref, outa_ref, outb_ref):
    xs = pf_ref[...] * _rs(do0_ref[...], do1_ref[...])
    outa_ref[...] = xs[:, 0:16]
    outb_ref[...] = jnp.pad(xs[:, 16:18], ((0, 0), (0, 14)))


def _tc_mid_body(aa0_ref, aa1_ref, ab0_ref, ab1_ref,
                 di0_ref, di1_ref, do0_ref, do1_ref,
                 w1_ref, w2_ref, o0_ref, o1_ref, o2_ref, o3_ref):
    a = jnp.concatenate([aa0_ref[0] + aa1_ref[0],
                         (ab0_ref[0] + ab1_ref[0])[:, 0:2]], axis=1)
    a = a * _rs(di0_ref[...], di1_ref[...])
    h1 = _lr(jnp.dot(a, w1_ref[...], preferred_element_type=jnp.float32, precision=lax.Precision.HIGHEST))
    p2 = jnp.dot(h1, w2_ref[...], preferred_element_type=jnp.float32, precision=lax.Precision.HIGHEST)
    p2 = p2 * _rs(do0_ref[...], do1_ref[...])
    o0_ref[...] = p2[:, 0:16]
    o1_ref[...] = p2[:, 16:32]
    o2_ref[...] = p2[:, 32:48]
    o3_ref[...] = p2[:, 48:64]


def _tc_pmean_body(b00, b01, b10, b11, b20, b21, b30, b31, di0, di1, out_ref):
    h2 = jnp.concatenate(
        [b00[0] + b01[0], b10[0] + b11[0],
         b20[0] + b21[0], b30[0] + b31[0]], axis=1)
    h2 = _lr(h2 * _rs(di0[...], di1[...]))
    out_ref[...] = jnp.mean(h2.reshape(5, NPP, 64), axis=1)[None]


def _tc_head_body(pm_ref, wlin_ref, wcls_ref, out_ref):
    r = jnp.maximum(pm_ref[...], 0.0)
    r = jnp.maximum(jnp.dot(r, wlin_ref[...], preferred_element_type=jnp.float32, precision=lax.Precision.HIGHEST), 0.0)
    out_ref[...] = jnp.dot(r, wcls_ref[...], preferred_element_type=jnp.float32, precision=lax.Precision.HIGHEST)


def _full(shape):
    return pl.BlockSpec(shape, lambda i: tuple(0 for _ in shape))


def _graph_conv(x, src, dst, n, W, ew=None):
    deg_out = jnp.clip(jnp.zeros((n,), jnp.float32).at[src].add(1.0), 1.0, None)
    deg_in = jnp.clip(jnp.zeros((n,), jnp.float32).at[dst].add(1.0), 1.0, None)
    h = x * (deg_out ** -0.5)[:, None]
    m = h[src]
    if ew is not None:
        m = m * ew[:, None]
    agg = jnp.zeros((n, x.shape[1]), x.dtype).at[dst].add(m)
    agg = agg * (deg_in ** -0.5)[:, None]
    return agg @ W


def kernel(patch_feats, patch_src, patch_dst, patch_edge_weight, patch_seg,
           mesh_src, mesh_dst, W_pc1, W_pc2, W_lin, W_cls, W_mc1, W_mc2, W_mcls):
    zeros_np = jnp.zeros((NP,), jnp.float32)
    dego, degi = _deg_kernel(patch_src, patch_dst, zeros_np)
    do0 = dego[0, :N].reshape(N, 1)
    do1 = dego[1, :N].reshape(N, 1)
    di0 = degi[0, :N].reshape(N, 1)
    di1 = degi[1, :N].reshape(N, 1)

    xs_a, xs_b = pl.pallas_call(
        _tc_xs_body,
        grid=(50,),
        in_specs=[pl.BlockSpec((2000, 18), lambda i: (i, 0)),
                  pl.BlockSpec((2000, 1), lambda i: (i, 0)),
                  pl.BlockSpec((2000, 1), lambda i: (i, 0))],
        out_specs=[pl.BlockSpec((2000, 16), lambda i: (i, 0)),
                   pl.BlockSpec((2000, 16), lambda i: (i, 0))],
        out_shape=[jax.ShapeDtypeStruct((N, 16), jnp.float32),
                   jax.ShapeDtypeStruct((N, 16), jnp.float32)],
    )(patch_feats, do0, do1)

    zeros16 = jnp.zeros((SLA, 16), jnp.float32)
    edges3 = jnp.stack([patch_src.astype(jnp.int32), patch_dst.astype(jnp.int32),
                        lax.bitcast_convert_type(patch_edge_weight, jnp.int32)])
    agg_a, agg_b = _agg16x2_kernel(xs_a, xs_b, edges3, zeros16)

    p2_chunks = pl.pallas_call(
        _tc_mid_body,
        grid=(50,),
        in_specs=[pl.BlockSpec((1, 2000, 16), lambda i: (0, i, 0)),
                  pl.BlockSpec((1, 2000, 16), lambda i: (1, i, 0)),
                  pl.BlockSpec((1, 2000, 16), lambda i: (0, i, 0)),
                  pl.BlockSpec((1, 2000, 16), lambda i: (1, i, 0)),
                  pl.BlockSpec((2000, 1), lambda i: (i, 0)),
                  pl.BlockSpec((2000, 1), lambda i: (i, 0)),
                  pl.BlockSpec((2000, 1), lambda i: (i, 0)),
                  pl.BlockSpec((2000, 1), lambda i: (i, 0)),
                  _full((18, 128)), _full((128, 64))],
        out_specs=[pl.BlockSpec((2000, 16), lambda i: (i, 0))] * 4,
        out_shape=[jax.ShapeDtypeStruct((N, 16), jnp.float32)] * 4,
    )(agg_a, agg_a, agg_b, agg_b, di0, di1, do0, do1, W_pc1, W_pc2)

    agg2 = _agg16x4_kernel(p2_chunks[0], p2_chunks[1], p2_chunks[2], p2_chunks[3],
                           edges3, zeros16)

    pm = pl.pallas_call(
        _tc_pmean_body,
        grid=(100,),
        in_specs=[pl.BlockSpec((1, 1000, 16), lambda i, c=c: (c, i, 0))
                  for _ in range(4) for c in range(2)]
                 + [pl.BlockSpec((1000, 1), lambda i: (i, 0)),
                    pl.BlockSpec((1000, 1), lambda i: (i, 0))],
        out_specs=pl.BlockSpec((1, 5, 64), lambda i: (i, 0, 0)),
        out_shape=jax.ShapeDtypeStruct((100, 5, 64), jnp.float32),
    )(agg2[0], agg2[0], agg2[1], agg2[1], agg2[2], agg2[2], agg2[3], agg2[3],
      di0, di1)
    pmean = pm.reshape(P, 64)

    readouts = pl.pallas_call(
        _tc_head_body,
        out_shape=jax.ShapeDtypeStruct((P, 16), jnp.float32),
    )(pmean, W_lin, W_cls)

    lr = lambda t: jax.nn.leaky_relu(t, 0.01)
    u = lr(_graph_conv(readouts, mesh_src, mesh_dst, MN, W_mc1))
    u = lr(_graph_conv(u, mesh_src, mesh_dst, MN, W_mc2))
    mesh_out = jnp.mean(u, axis=0, keepdims=True) @ W_mcls
    return (mesh_out, readouts)


# speedup vs baseline: 1.1427x; 1.1427x over previous
import dataclasses
import functools

import jax
import jax.numpy as jnp
from jax import lax
from jax.experimental import pallas as pl
from jax.experimental.pallas import tpu as pltpu
from jax.experimental.pallas import tpu_sc as plsc

N = 100000
NP = 102400
P = 500
NPP = 200
E = 1600000
MN = 500

NC = 2
NS = 16
NW = NC * NS
EW_PER = E // NW
CHUNK = 2000
SLN = NP // NS
NA = 100096
SLA = NA // NS
CH1 = 400
CH2 = 400
EW_MAIN = (EW_PER // (4 * CH2)) * (4 * CH2)
assert EW_PER - EW_MAIN == CH2

_mesh = plsc.VectorSubcoreMesh(core_axis_name="c", subcore_axis_name="s")

_sc_params = pltpu.CompilerParams(needs_layout_passes=False,
                                  use_tc_tiling_on_sc=False)


@functools.partial(
    pl.kernel,
    out_type=(jax.ShapeDtypeStruct((NC, NP), jnp.float32),
              jax.ShapeDtypeStruct((NC, NP), jnp.float32)),
    mesh=_mesh,
    scratch_types=[
        pltpu.VMEM((CHUNK,), jnp.int32),
        pltpu.VMEM((CHUNK,), jnp.int32),
        pltpu.VMEM((CHUNK,), jnp.float32),
        pltpu.VMEM_SHARED((NP,), jnp.float32),
        pltpu.VMEM_SHARED((NP,), jnp.float32),
    ],
)
def _deg_kernel(src_hbm, dst_hbm, zeros_hbm, dego_hbm, degi_hbm,
                src_v, dst_v, ones_v, acco_s, acci_s):
    cid = lax.axis_index("c")
    sid = lax.axis_index("s")
    wid = cid * NS + sid

    @pl.loop(0, CHUNK, step=16)
    def _(i):
        ones_v[pl.ds(pl.multiple_of(i, 16), 16)] = jnp.ones((16,), jnp.float32)

    pltpu.sync_copy(zeros_hbm.at[pl.ds(sid * SLN, SLN)], acco_s.at[pl.ds(sid * SLN, SLN)])
    pltpu.sync_copy(zeros_hbm.at[pl.ds(sid * SLN, SLN)], acci_s.at[pl.ds(sid * SLN, SLN)])
    plsc.subcore_barrier()

    @pl.loop(0, EW_PER, step=CHUNK)
    def _(k):
        base = wid * EW_PER + k
        pltpu.sync_copy(src_hbm.at[pl.ds(base, CHUNK)], src_v)
        pltpu.sync_copy(dst_hbm.at[pl.ds(base, CHUNK)], dst_v)
        pltpu.sync_copy(ones_v, acco_s.at[src_v], add=True)
        pltpu.sync_copy(ones_v, acci_s.at[dst_v], add=True)

    plsc.subcore_barrier()
    pltpu.sync_copy(acco_s.at[pl.ds(sid * SLN, SLN)], dego_hbm.at[cid, pl.ds(sid * SLN, SLN)])
    pltpu.sync_copy(acci_s.at[pl.ds(sid * SLN, SLN)], degi_hbm.at[cid, pl.ds(sid * SLN, SLN)])


def _scale_rows(rows_v, ew_v, iota16, F, chunk):
    @pl.loop(0, chunk, step=16)
    def _(j):
        e16 = jnp.full((16,), j, jnp.int32) + iota16
        w16 = ew_v[pl.ds(pl.multiple_of(j, 16), 16)]
        for f in range(F):
            fv = jnp.full((16,), f, jnp.int32)
            v = plsc.load_gather(rows_v, [e16, fv])
            plsc.store_scatter(rows_v, [e16, fv], v * w16)


def _make_agg_kernel(F, n_chunks):
    n_sets = 4
    scratch = []
    for _ in range(n_sets):
        scratch += [pltpu.VMEM((CH2,), jnp.int32), pltpu.VMEM((CH2,), jnp.int32),
                    pltpu.VMEM((CH2,), jnp.float32), pltpu.VMEM((CH2, F), jnp.float32),
                    pltpu.SemaphoreType.DMA, pltpu.SemaphoreType.DMA]
    scratch.append(pltpu.VMEM_SHARED((NA, F), jnp.float32))

    @functools.partial(
        pl.kernel,
        out_type=tuple(jax.ShapeDtypeStruct((NC, NA, F), jnp.float32)
                       for _ in range(n_chunks)),
        mesh=_mesh,
        scratch_types=scratch,
        compiler_params=_sc_params,
    )
    def _agg(*refs):
        xs_list = refs[:n_chunks]
        src_hbm, dst_hbm, ew_hbm, zeros_hbm = refs[n_chunks:n_chunks + 4]
        out_list = refs[n_chunks + 4:2 * n_chunks + 4]
        rest = refs[2 * n_chunks + 4:]
        sets = [tuple(rest[6 * i:6 * i + 6]) for i in range(n_sets)]
        acc_s = rest[6 * n_sets]

        cid = lax.axis_index("c")
        sid = lax.axis_index("s")
        wid = cid * NS + sid
        iota16 = lax.iota(jnp.int32, 16)
        step = n_sets * CH2

        def chain(set_refs, base, c, first):
            src_v, dst_v, ew_v, rows_v, g_sem, sc_sem = set_refs
            if first is None:
                pltpu.make_async_copy(rows_v, acc_s.at[dst_v], sc_sem).wait()
            else:
                @pl.when(first)
                def _():
                    pltpu.make_async_copy(rows_v, acc_s.at[dst_v], sc_sem).wait()
            return (pltpu.async_copy(src_hbm.at[pl.ds(base, CH2)], src_v, g_sem),
                    pltpu.async_copy(dst_hbm.at[pl.ds(base, CH2)], dst_v, g_sem),
                    pltpu.async_copy(ew_hbm.at[pl.ds(base, CH2)], ew_v, g_sem))

        for c in range(n_chunks):
            pltpu.sync_copy(zeros_hbm, acc_s.at[pl.ds(sid * SLA, SLA)])
            plsc.subcore_barrier()

            @pl.loop(0, EW_MAIN, step=step)
            def _(k, c=c):
                idx_d = [chain(sets[si], wid * EW_PER + k + si * CH2, c, k > 0)
                         for si in range(n_sets)]
                gat_d = []
                for si in range(n_sets):
                    src_v, dst_v, ew_v, rows_v, g_sem, sc_sem = sets[si]
                    for d in idx_d[si]:
                        d.wait()
                    gat_d.append(pltpu.async_copy(xs_list[c].at[src_v], rows_v,
                                                  g_sem))
                for si in range(n_sets):
                    src_v, dst_v, ew_v, rows_v, g_sem, sc_sem = sets[si]
                    gat_d[si].wait()
                    _scale_rows(rows_v, ew_v, iota16, F, CH2)
                    pltpu.async_copy(rows_v, acc_s.at[dst_v], sc_sem, add=True)

            src_v, dst_v, ew_v, rows_v, g_sem, sc_sem = sets[0]
            for d in chain(sets[0], wid * EW_PER + EW_MAIN, c, None):
                d.wait()
            pltpu.async_copy(xs_list[c].at[src_v], rows_v, g_sem).wait()
            _scale_rows(rows_v, ew_v, iota16, F, CH2)
            pltpu.async_copy(rows_v, acc_s.at[dst_v], sc_sem, add=True)

            for si in range(n_sets):
                src_v, dst_v, ew_v, rows_v, g_sem, sc_sem = sets[si]
                pltpu.make_async_copy(rows_v, acc_s.at[dst_v], sc_sem).wait()

            plsc.subcore_barrier()
            pltpu.sync_copy(acc_s.at[pl.ds(sid * SLA, SLA)],
                            out_list[c].at[cid, pl.ds(sid * SLA, SLA)])

    return _agg


_agg16x4_kernel = _make_agg_kernel(16, 4)
_agg16x2_kernel = _make_agg_kernel(16, 2)


def _lr(t):
    return jnp.where(t >= 0.0, t, 0.01 * t)


def _rs(d0, d1):
    return lax.rsqrt(jnp.maximum(d0 + d1, 1.0))


def _tc_xs_body(pf_ref, do0_ref, do1_ref, outa_ref, outb_ref):
    xs = pf_ref[...] * _rs(do0_ref[...], do1_ref[...])
    outa_ref[...] = xs[:, 0:16]
    outb_ref[...] = jnp.pad(xs[:, 16:18], ((0, 0), (0, 14)))


def _tc_mid_body(aa0_ref, aa1_ref, ab0_ref, ab1_ref,
                 di0_ref, di1_ref, do0_ref, do1_ref,
                 w1_ref, w2_ref, o0_ref, o1_ref, o2_ref, o3_ref):
    a = jnp.concatenate([aa0_ref[0] + aa1_ref[0],
                         (ab0_ref[0] + ab1_ref[0])[:, 0:2]], axis=1)
    a = a * _rs(di0_ref[...], di1_ref[...])
    h1 = _lr(jnp.dot(a, w1_ref[...], preferred_element_type=jnp.float32, precision=lax.Precision.HIGHEST))
    p2 = jnp.dot(h1, w2_ref[...], preferred_element_type=jnp.float32, precision=lax.Precision.HIGHEST)
    p2 = p2 * _rs(do0_ref[...], do1_ref[...])
    o0_ref[...] = p2[:, 0:16]
    o1_ref[...] = p2[:, 16:32]
    o2_ref[...] = p2[:, 32:48]
    o3_ref[...] = p2[:, 48:64]


def _tc_pmean_body(b00, b01, b10, b11, b20, b21, b30, b31, di0, di1, out_ref):
    h2 = jnp.concatenate(
        [b00[0] + b01[0], b10[0] + b11[0],
         b20[0] + b21[0], b30[0] + b31[0]], axis=1)
    h2 = _lr(h2 * _rs(di0[...], di1[...]))
    out_ref[...] = jnp.mean(h2.reshape(5, NPP, 64), axis=1)[None]


def _tc_head_body(pm_ref, wlin_ref, wcls_ref, out_ref):
    r = jnp.maximum(pm_ref[...], 0.0)
    r = jnp.maximum(jnp.dot(r, wlin_ref[...], preferred_element_type=jnp.float32, precision=lax.Precision.HIGHEST), 0.0)
    out_ref[...] = jnp.dot(r, wcls_ref[...], preferred_element_type=jnp.float32, precision=lax.Precision.HIGHEST)


def _full(shape):
    return pl.BlockSpec(shape, lambda i: tuple(0 for _ in shape))


def _graph_conv(x, src, dst, n, W, ew=None):
    deg_out = jnp.clip(jnp.zeros((n,), jnp.float32).at[src].add(1.0), 1.0, None)
    deg_in = jnp.clip(jnp.zeros((n,), jnp.float32).at[dst].add(1.0), 1.0, None)
    h = x * (deg_out ** -0.5)[:, None]
    m = h[src]
    if ew is not None:
        m = m * ew[:, None]
    agg = jnp.zeros((n, x.shape[1]), x.dtype).at[dst].add(m)
    agg = agg * (deg_in ** -0.5)[:, None]
    return agg @ W


def kernel(patch_feats, patch_src, patch_dst, patch_edge_weight, patch_seg,
           mesh_src, mesh_dst, W_pc1, W_pc2, W_lin, W_cls, W_mc1, W_mc2, W_mcls):
    zeros_np = jnp.zeros((NP,), jnp.float32)
    dego, degi = _deg_kernel(patch_src, patch_dst, zeros_np)
    do0 = dego[0, :N].reshape(N, 1)
    do1 = dego[1, :N].reshape(N, 1)
    di0 = degi[0, :N].reshape(N, 1)
    di1 = degi[1, :N].reshape(N, 1)

    xs_a, xs_b = pl.pallas_call(
        _tc_xs_body,
        grid=(50,),
        in_specs=[pl.BlockSpec((2000, 18), lambda i: (i, 0)),
                  pl.BlockSpec((2000, 1), lambda i: (i, 0)),
                  pl.BlockSpec((2000, 1), lambda i: (i, 0))],
        out_specs=[pl.BlockSpec((2000, 16), lambda i: (i, 0)),
                   pl.BlockSpec((2000, 16), lambda i: (i, 0))],
        out_shape=[jax.ShapeDtypeStruct((N, 16), jnp.float32),
                   jax.ShapeDtypeStruct((N, 16), jnp.float32)],
    )(patch_feats, do0, do1)

    zeros16 = jnp.zeros((SLA, 16), jnp.float32)
    agg_a, agg_b = _agg16x2_kernel(xs_a, xs_b, patch_src, patch_dst,
                                   patch_edge_weight, zeros16)

    p2_chunks = pl.pallas_call(
        _tc_mid_body,
        grid=(50,),
        in_specs=[pl.BlockSpec((1, 2000, 16), lambda i: (0, i, 0)),
                  pl.BlockSpec((1, 2000, 16), lambda i: (1, i, 0)),
                  pl.BlockSpec((1, 2000, 16), lambda i: (0, i, 0)),
                  pl.BlockSpec((1, 2000, 16), lambda i: (1, i, 0)),
                  pl.BlockSpec((2000, 1), lambda i: (i, 0)),
                  pl.BlockSpec((2000, 1), lambda i: (i, 0)),
                  pl.BlockSpec((2000, 1), lambda i: (i, 0)),
                  pl.BlockSpec((2000, 1), lambda i: (i, 0)),
                  _full((18, 128)), _full((128, 64))],
        out_specs=[pl.BlockSpec((2000, 16), lambda i: (i, 0))] * 4,
        out_shape=[jax.ShapeDtypeStruct((N, 16), jnp.float32)] * 4,
    )(agg_a, agg_a, agg_b, agg_b, di0, di1, do0, do1, W_pc1, W_pc2)

    agg2 = _agg16x4_kernel(p2_chunks[0], p2_chunks[1], p2_chunks[2], p2_chunks[3],
                           patch_src, patch_dst, patch_edge_weight, zeros16)

    pm = pl.pallas_call(
        _tc_pmean_body,
        grid=(100,),
        in_specs=[pl.BlockSpec((1, 1000, 16), lambda i, c=c: (c, i, 0))
                  for _ in range(4) for c in range(2)]
                 + [pl.BlockSpec((1000, 1), lambda i: (i, 0)),
                    pl.BlockSpec((1000, 1), lambda i: (i, 0))],
        out_specs=pl.BlockSpec((1, 5, 64), lambda i: (i, 0, 0)),
        out_shape=jax.ShapeDtypeStruct((100, 5, 64), jnp.float32),
    )(agg2[0], agg2[0], agg2[1], agg2[1], agg2[2], agg2[2], agg2[3], agg2[3],
      di0, di1)
    pmean = pm.reshape(P, 64)

    readouts = pl.pallas_call(
        _tc_head_body,
        out_shape=jax.ShapeDtypeStruct((P, 16), jnp.float32),
    )(pmean, W_lin, W_cls)

    lr = lambda t: jax.nn.leaky_relu(t, 0.01)
    u = lr(_graph_conv(readouts, mesh_src, mesh_dst, MN, W_mc1))
    u = lr(_graph_conv(u, mesh_src, mesh_dst, MN, W_mc2))
    mesh_out = jnp.mean(u, axis=0, keepdims=True) @ W_mcls
    return (mesh_out, readouts)
